# baseline (speedup 1.0000x reference)
import functools

import jax
import jax.numpy as jnp
from jax import lax
from jax.experimental import pallas as pl
from jax.experimental.pallas import tpu as pltpu
from jax.experimental.pallas import tpu_sc as plsc

N_KEYS = 100000
BATCH = 1024
DIM = 64
KBLK = 2000
NBLK = N_KEYS // KBLK

NCORE = 2
NSUB = 16
ROWS_PER_SUB = 6256
ROWS_LAST = N_KEYS - 15 * ROWS_PER_SUB
AGE_PAD = 102400
AGE_PER_SUB = AGE_PAD // NSUB
UPD_PER_SUB = BATCH // NSUB


def _attn_body(q_ref, vt_ref, g_ref, b_ref, k_ref, v_ref,
               ret_ref, sur_ref, qn_ref, dec_ref,
               m_ref, l_ref, acc_ref):
    i = pl.program_id(0)

    @pl.when(i == 0)
    def _init():
        q = q_ref[...]
        mu = jnp.mean(q, axis=1, keepdims=True)
        var = jnp.mean((q - mu) ** 2, axis=1, keepdims=True)
        qn = (q - mu) * lax.rsqrt(var + 1e-5) * g_ref[...] + b_ref[...]
        qn_ref[...] = qn
        m_ref[...] = jnp.full((BATCH, 1), -1e30, jnp.float32)
        l_ref[...] = jnp.zeros((BATCH, 1), jnp.float32)
        acc_ref[...] = jnp.zeros((BATCH, DIM), jnp.float32)

    qn = qn_ref[...]
    qsq = jnp.sum(qn * qn, axis=1, keepdims=True)
    k = k_ref[...]
    kk = jnp.sum(k * k, axis=1)[None, :]
    qk = lax.dot_general(qn, k, (((1,), (1,)), ((), ())),
                         preferred_element_type=jnp.float32)
    s = -jnp.maximum(qsq + kk - 2.0 * qk, 0.0)
    m_prev = m_ref[...]
    m_new = jnp.maximum(m_prev, jnp.max(s, axis=1, keepdims=True))
    alpha = jnp.exp(m_prev - m_new)
    p = jnp.exp(s - m_new)
    l_ref[...] = l_ref[...] * alpha + jnp.sum(p, axis=1, keepdims=True)
    acc_ref[...] = acc_ref[...] * alpha + lax.dot_general(
        p, v_ref[...], (((1,), (0,)), ((), ())),
        preferred_element_type=jnp.float32)
    m_ref[...] = m_new

    @pl.when(i == NBLK - 1)
    def _fin():
        r = acc_ref[...] / l_ref[...]
        ret_ref[...] = r
        diff = r - vt_ref[...]
        sur = jnp.mean(diff * diff, axis=1, keepdims=True)
        sur_ref[...] = sur
        w = jax.nn.sigmoid(sur - jnp.mean(sur))
        dec_ref[...] = 0.99 * (1.0 - w)


def _attention(query, value_target, keys, values, gamma, beta):
    return pl.pallas_call(
        _attn_body,
        grid=(NBLK,),
        in_specs=[
            pl.BlockSpec((BATCH, DIM), lambda i: (0, 0)),
            pl.BlockSpec((BATCH, DIM), lambda i: (0, 0)),
            pl.BlockSpec((1, DIM), lambda i: (0, 0)),
            pl.BlockSpec((1, DIM), lambda i: (0, 0)),
            pl.BlockSpec((KBLK, DIM), lambda i: (i, 0)),
            pl.BlockSpec((KBLK, DIM), lambda i: (i, 0)),
        ],
        out_specs=[
            pl.BlockSpec((BATCH, DIM), lambda i: (0, 0)),
            pl.BlockSpec((BATCH, 1), lambda i: (0, 0)),
            pl.BlockSpec((BATCH, DIM), lambda i: (0, 0)),
            pl.BlockSpec((BATCH, 1), lambda i: (0, 0)),
        ],
        out_shape=[
            jax.ShapeDtypeStruct((BATCH, DIM), jnp.float32),
            jax.ShapeDtypeStruct((BATCH, 1), jnp.float32),
            jax.ShapeDtypeStruct((BATCH, DIM), jnp.float32),
            jax.ShapeDtypeStruct((BATCH, 1), jnp.float32),
        ],
        scratch_shapes=[
            pltpu.VMEM((BATCH, 1), jnp.float32),
            pltpu.VMEM((BATCH, 1), jnp.float32),
            pltpu.VMEM((BATCH, DIM), jnp.float32),
        ],
        compiler_params=pltpu.CompilerParams(
            dimension_semantics=("arbitrary",),
        ),
    )(query, value_target, gamma.reshape(1, DIM), beta.reshape(1, DIM),
      keys, values)


def _blend_rows(rows_v, src_v, decay_v):
    def body(g, carry):
        dgrp = decay_v[pl.ds(g * 16, 16)]
        for r16 in range(16):
            r = g * 16 + r16
            d = jnp.full((16,), dgrp[r16], jnp.float32)
            omd = 1.0 - d
            for c in range(DIM // 16):
                sl = pl.ds(c * 16, 16)
                rows_v[r, sl] = d * rows_v[r, sl] + omd * src_v[r, sl]
        return carry
    lax.fori_loop(0, UPD_PER_SUB // 16, body, 0)


def _write_body(keys_hbm, values_hbm, age_hbm, idx_hbm, decay_hbm, qn_hbm,
                vt_hbm, out_k, out_v, out_a,
                idx_v, decay_v, rows_v, src_v, age_v, sem):
    cid = lax.axis_index("c")
    sid = lax.axis_index("s")

    rbase = sid * ROWS_PER_SUB

    @pl.when(cid == 0)
    def _copy_keys_age():
        @pl.when(sid < NSUB - 1)
        def _():
            pltpu.sync_copy(keys_hbm.at[pl.ds(rbase, ROWS_PER_SUB), :],
                            out_k.at[pl.ds(rbase, ROWS_PER_SUB), :])

        @pl.when(sid == NSUB - 1)
        def _():
            pltpu.sync_copy(keys_hbm.at[pl.ds(rbase, ROWS_LAST), :],
                            out_k.at[pl.ds(rbase, ROWS_LAST), :])
        abase = sid * AGE_PER_SUB
        pltpu.sync_copy(age_hbm.at[pl.ds(abase, AGE_PER_SUB)], age_v)

        def aging(i, carry):
            sl = pl.ds(i * 16, 16)
            age_v[sl] = age_v[sl] + 1.0
            return carry
        lax.fori_loop(0, AGE_PER_SUB // 16, aging, 0)
        pltpu.sync_copy(age_v, out_a.at[pl.ds(abase, AGE_PER_SUB)])

    @pl.when(cid == 1)
    def _copy_values():
        @pl.when(sid < NSUB - 1)
        def _():
            pltpu.sync_copy(values_hbm.at[pl.ds(rbase, ROWS_PER_SUB), :],
                            out_v.at[pl.ds(rbase, ROWS_PER_SUB), :])

        @pl.when(sid == NSUB - 1)
        def _():
            pltpu.sync_copy(values_hbm.at[pl.ds(rbase, ROWS_LAST), :],
                            out_v.at[pl.ds(rbase, ROWS_LAST), :])

    plsc.subcore_barrier()

    ubase = sid * UPD_PER_SUB
    pltpu.sync_copy(idx_hbm.at[pl.ds(ubase, UPD_PER_SUB)], idx_v)
    pltpu.sync_copy(decay_hbm.at[pl.ds(ubase, UPD_PER_SUB)], decay_v)

    @pl.when(cid == 0)
    def _update_keys_age():
        pltpu.async_copy(keys_hbm.at[idx_v], rows_v, sem).wait()
        pltpu.sync_copy(qn_hbm.at[pl.ds(ubase, UPD_PER_SUB), :], src_v)
        _blend_rows(rows_v, src_v, decay_v)
        pltpu.async_copy(rows_v, out_k.at[idx_v], sem).wait()
        for c in range(UPD_PER_SUB // 16):
            age_v[pl.ds(c * 16, 16)] = jnp.full((16,), 1.0, jnp.float32)
        pltpu.async_copy(age_v.at[pl.ds(0, UPD_PER_SUB)], out_a.at[idx_v],
                         sem).wait()

    @pl.when(cid == 1)
    def _update_values():
        pltpu.async_copy(values_hbm.at[idx_v], rows_v, sem).wait()
        pltpu.sync_copy(vt_hbm.at[pl.ds(ubase, UPD_PER_SUB), :], src_v)
        _blend_rows(rows_v, src_v, decay_v)
        pltpu.async_copy(rows_v, out_v.at[idx_v], sem).wait()


_write = functools.partial(
    pl.kernel,
    out_type=[
        jax.ShapeDtypeStruct((N_KEYS, DIM), jnp.float32),
        jax.ShapeDtypeStruct((N_KEYS, DIM), jnp.float32),
        jax.ShapeDtypeStruct((AGE_PAD,), jnp.float32),
    ],
    mesh=plsc.VectorSubcoreMesh(core_axis_name="c", subcore_axis_name="s"),
    scratch_types=[
        pltpu.VMEM((UPD_PER_SUB,), jnp.int32),
        pltpu.VMEM((UPD_PER_SUB,), jnp.float32),
        pltpu.VMEM((UPD_PER_SUB, DIM), jnp.float32),
        pltpu.VMEM((UPD_PER_SUB, DIM), jnp.float32),
        pltpu.VMEM((AGE_PER_SUB,), jnp.float32),
        pltpu.SemaphoreType.DMA,
    ],
    compiler_params=pltpu.CompilerParams(use_tc_tiling_on_sc=False),
)(_write_body)


def kernel(query, value_target, keys, values, slot_age, kn_gamma, kn_beta):
    retrieved, sur, qn, dec = _attention(
        query, value_target, keys, values, kn_gamma, kn_beta)
    surprise = sur[:, 0]
    decay = dec[:, 0]
    _, oldest = lax.top_k(slot_age, BATCH)
    age_pad = jnp.pad(slot_age, (0, AGE_PAD - N_KEYS))
    new_keys, new_values, new_age = _write(
        keys, values, age_pad, oldest, decay, qn, value_target)
    return retrieved, surprise, new_keys, new_values, new_age[:N_KEYS]


# speedup vs baseline: 3.1654x; 3.1654x over previous
import functools

import jax
import jax.numpy as jnp
from jax import lax
from jax.experimental import pallas as pl
from jax.experimental.pallas import tpu as pltpu
from jax.experimental.pallas import tpu_sc as plsc

N_KEYS = 100000
BATCH = 1024
DIM = 64
KBLK = 2000
NBLK = N_KEYS // KBLK

NCORE = 2
NSUB = 16
ROWS_PER_SUB = 6256
ROWS_LAST = N_KEYS - 15 * ROWS_PER_SUB
AGE_PAD = 102400
AGE_PER_SUB = AGE_PAD // NSUB
UPD_PER_SUB = BATCH // NSUB


def _attn_body(q_ref, vt_ref, g_ref, b_ref, k_ref, v_ref,
               ret_ref, sur_ref, qn_ref, dec_ref, kc_ref, vc_ref,
               m_ref, l_ref, acc_ref):
    i = pl.program_id(0)
    kc_ref[...] = k_ref[...]
    vc_ref[...] = v_ref[...]

    @pl.when(i == 0)
    def _init():
        q = q_ref[...]
        mu = jnp.mean(q, axis=1, keepdims=True)
        var = jnp.mean((q - mu) ** 2, axis=1, keepdims=True)
        qn = (q - mu) * lax.rsqrt(var + 1e-5) * g_ref[...] + b_ref[...]
        qn_ref[...] = qn
        m_ref[...] = jnp.full((BATCH, 1), -1e30, jnp.float32)
        l_ref[...] = jnp.zeros((BATCH, 1), jnp.float32)
        acc_ref[...] = jnp.zeros((BATCH, DIM), jnp.float32)

    qn = qn_ref[...]
    qsq = jnp.sum(qn * qn, axis=1, keepdims=True)
    k = k_ref[...]
    kk = jnp.sum(k * k, axis=1)[None, :]
    qk = lax.dot_general(qn, k, (((1,), (1,)), ((), ())),
                         preferred_element_type=jnp.float32)
    s = -jnp.maximum(qsq + kk - 2.0 * qk, 0.0)
    m_prev = m_ref[...]
    m_new = jnp.maximum(m_prev, jnp.max(s, axis=1, keepdims=True))
    alpha = jnp.exp(m_prev - m_new)
    p = jnp.exp(s - m_new)
    l_ref[...] = l_ref[...] * alpha + jnp.sum(p, axis=1, keepdims=True)
    acc_ref[...] = acc_ref[...] * alpha + lax.dot_general(
        p, v_ref[...], (((1,), (0,)), ((), ())),
        preferred_element_type=jnp.float32)
    m_ref[...] = m_new

    @pl.when(i == NBLK - 1)
    def _fin():
        r = acc_ref[...] / l_ref[...]
        ret_ref[...] = r
        diff = r - vt_ref[...]
        sur = jnp.mean(diff * diff, axis=1, keepdims=True)
        sur_ref[...] = sur
        w = jax.nn.sigmoid(sur - jnp.mean(sur))
        dec_ref[...] = 0.99 * (1.0 - w)


def _attention(query, value_target, keys, values, gamma, beta):
    return pl.pallas_call(
        _attn_body,
        grid=(NBLK,),
        in_specs=[
            pl.BlockSpec((BATCH, DIM), lambda i: (0, 0)),
            pl.BlockSpec((BATCH, DIM), lambda i: (0, 0)),
            pl.BlockSpec((1, DIM), lambda i: (0, 0)),
            pl.BlockSpec((1, DIM), lambda i: (0, 0)),
            pl.BlockSpec((KBLK, DIM), lambda i: (i, 0)),
            pl.BlockSpec((KBLK, DIM), lambda i: (i, 0)),
        ],
        out_specs=[
            pl.BlockSpec((BATCH, DIM), lambda i: (0, 0)),
            pl.BlockSpec((BATCH, 1), lambda i: (0, 0)),
            pl.BlockSpec((BATCH, DIM), lambda i: (0, 0)),
            pl.BlockSpec((BATCH, 1), lambda i: (0, 0)),
            pl.BlockSpec((KBLK, DIM), lambda i: (i, 0)),
            pl.BlockSpec((KBLK, DIM), lambda i: (i, 0)),
        ],
        out_shape=[
            jax.ShapeDtypeStruct((BATCH, DIM), jnp.float32),
            jax.ShapeDtypeStruct((BATCH, 1), jnp.float32),
            jax.ShapeDtypeStruct((BATCH, DIM), jnp.float32),
            jax.ShapeDtypeStruct((BATCH, 1), jnp.float32),
            jax.ShapeDtypeStruct((N_KEYS, DIM), jnp.float32),
            jax.ShapeDtypeStruct((N_KEYS, DIM), jnp.float32),
        ],
        scratch_shapes=[
            pltpu.VMEM((BATCH, 1), jnp.float32),
            pltpu.VMEM((BATCH, 1), jnp.float32),
            pltpu.VMEM((BATCH, DIM), jnp.float32),
        ],
        compiler_params=pltpu.CompilerParams(
            dimension_semantics=("arbitrary",),
        ),
    )(query, value_target, gamma.reshape(1, DIM), beta.reshape(1, DIM),
      keys, values)


def _blend_rows(rows_v, src_v, decay_v):
    def body(g, carry):
        dgrp = decay_v[pl.ds(g * 16, 16)]
        for r16 in range(16):
            r = g * 16 + r16
            d = jnp.full((16,), dgrp[r16], jnp.float32)
            omd = 1.0 - d
            for c in range(DIM // 16):
                sl = pl.ds(c * 16, 16)
                rows_v[r, sl] = d * rows_v[r, sl] + omd * src_v[r, sl]
        return carry
    lax.fori_loop(0, UPD_PER_SUB // 16, body, 0)


def _write_body(kc, vc, age_hbm, idx_hbm, decay_hbm, qn_hbm, vt_hbm,
                out_a, idx_v, decay_v, rows_v, src_v, age_v, sem):
    cid = lax.axis_index("c")
    sid = lax.axis_index("s")

    @pl.when(cid == 0)
    def _age():
        abase = sid * AGE_PER_SUB
        pltpu.sync_copy(age_hbm.at[pl.ds(abase, AGE_PER_SUB)], age_v)

        def aging(i, carry):
            sl = pl.ds(i * 16, 16)
            age_v[sl] = age_v[sl] + 1.0
            return carry
        lax.fori_loop(0, AGE_PER_SUB // 16, aging, 0)
        pltpu.sync_copy(age_v, out_a.at[pl.ds(abase, AGE_PER_SUB)])

    plsc.subcore_barrier()

    ubase = sid * UPD_PER_SUB
    pltpu.sync_copy(idx_hbm.at[pl.ds(ubase, UPD_PER_SUB)], idx_v)
    pltpu.sync_copy(decay_hbm.at[pl.ds(ubase, UPD_PER_SUB)], decay_v)

    @pl.when(cid == 0)
    def _update_keys_age():
        pltpu.async_copy(kc.at[idx_v], rows_v, sem).wait()
        pltpu.sync_copy(qn_hbm.at[pl.ds(ubase, UPD_PER_SUB), :], src_v)
        _blend_rows(rows_v, src_v, decay_v)
        pltpu.async_copy(rows_v, kc.at[idx_v], sem).wait()
        for c in range(UPD_PER_SUB // 16):
            age_v[pl.ds(c * 16, 16)] = jnp.full((16,), 1.0, jnp.float32)
        pltpu.async_copy(age_v.at[pl.ds(0, UPD_PER_SUB)], out_a.at[idx_v],
                         sem).wait()

    @pl.when(cid == 1)
    def _update_values():
        pltpu.async_copy(vc.at[idx_v], rows_v, sem).wait()
        pltpu.sync_copy(vt_hbm.at[pl.ds(ubase, UPD_PER_SUB), :], src_v)
        _blend_rows(rows_v, src_v, decay_v)
        pltpu.async_copy(rows_v, vc.at[idx_v], sem).wait()


_write = functools.partial(
    pl.kernel,
    out_type=jax.ShapeDtypeStruct((AGE_PAD,), jnp.float32),
    mesh=plsc.VectorSubcoreMesh(core_axis_name="c", subcore_axis_name="s"),
    scratch_types=[
        pltpu.VMEM((UPD_PER_SUB,), jnp.int32),
        pltpu.VMEM((UPD_PER_SUB,), jnp.float32),
        pltpu.VMEM((UPD_PER_SUB, DIM), jnp.float32),
        pltpu.VMEM((UPD_PER_SUB, DIM), jnp.float32),
        pltpu.VMEM((AGE_PER_SUB,), jnp.float32),
        pltpu.SemaphoreType.DMA,
    ],
    compiler_params=pltpu.CompilerParams(use_tc_tiling_on_sc=False),
)(_write_body)


def kernel(query, value_target, keys, values, slot_age, kn_gamma, kn_beta):
    retrieved, sur, qn, dec, kcopy, vcopy = _attention(
        query, value_target, keys, values, kn_gamma, kn_beta)
    surprise = sur[:, 0]
    decay = dec[:, 0]
    _, oldest = lax.top_k(slot_age, BATCH)
    age_pad = jnp.pad(slot_age, (0, AGE_PAD - N_KEYS))
    kc_ref = jax.new_ref(kcopy)
    vc_ref = jax.new_ref(vcopy)
    new_age = _write(kc_ref, vc_ref, age_pad, oldest, decay, qn, value_target)
    return retrieved, surprise, kc_ref[...], vc_ref[...], new_age[:N_KEYS]
